# baseline (speedup 1.0000x reference)
import jax
import jax.numpy as jnp
from jax.experimental import pallas as pl
from jax.experimental.pallas import tpu as pltpu

_F32 = jnp.float32
_BF16 = jnp.bfloat16


def _b(v):
    return v.astype(_BF16)


def _embed_body(x0, x1, x2, w0, b0, w1, b1, w2, b2, h0, h1, h2):
    h0[...] = jnp.dot(_b(x0[...]), _b(w0[...]), preferred_element_type=_F32) + b0[...]
    h1[...] = jnp.dot(_b(x1[...]), _b(w1[...]), preferred_element_type=_F32) + b1[...]
    h2[...] = jnp.dot(_b(x2[...]), _b(w2[...]), preferred_element_type=_F32) + b2[...]


def _embed(x0, x1, x2, W0, b0, W1, b1, W2, b2):
    C = W0.shape[1]
    outs = [jax.ShapeDtypeStruct((x.shape[0], C), _F32) for x in (x0, x1, x2)]
    return pl.pallas_call(_embed_body, out_shape=outs)(
        x0, x1, x2, W0, b0.reshape(1, -1), W1, b1.reshape(1, -1), W2, b2.reshape(1, -1)
    )


def _inc_body(B_ref, xs_ref, xd_ref, f_ref, bwd_ref):
    i = pl.program_id(0)
    blk = _b(B_ref[...])
    f_ref[...] = jnp.dot(blk, _b(xs_ref[...]), preferred_element_type=_F32)

    @pl.when(i == 0)
    def _():
        bwd_ref[...] = jnp.zeros_like(bwd_ref)

    bwd_ref[...] += jax.lax.dot_general(
        blk, _b(xd_ref[...]), dimension_numbers=(((0,), (0,)), ((), ())),
        preferred_element_type=_F32)


def _inc_fwd_body(B_ref, xs_ref, f_ref):
    f_ref[...] = jnp.dot(_b(B_ref[...]), _b(xs_ref[...]),
                         preferred_element_type=_F32)


def _incidence_fwd(B, xs, R=512):
    Nr, Nc = B.shape
    C = xs.shape[1]
    return pl.pallas_call(
        _inc_fwd_body,
        grid=(Nr // R,),
        in_specs=[
            pl.BlockSpec((R, Nc), lambda i: (i, 0)),
            pl.BlockSpec((Nc, C), lambda i: (0, 0)),
        ],
        out_specs=pl.BlockSpec((R, C), lambda i: (i, 0)),
        out_shape=jax.ShapeDtypeStruct((Nr, C), _F32),
        compiler_params=pltpu.CompilerParams(dimension_semantics=("arbitrary",)),
    )(B, xs)


def _incidence(B, xs, xd, R=256):
    Nr, Nc = B.shape
    C = xs.shape[1]
    return pl.pallas_call(
        _inc_body,
        grid=(Nr // R,),
        in_specs=[
            pl.BlockSpec((R, Nc), lambda i: (i, 0)),
            pl.BlockSpec((Nc, C), lambda i: (0, 0)),
            pl.BlockSpec((R, C), lambda i: (i, 0)),
        ],
        out_specs=[
            pl.BlockSpec((R, C), lambda i: (i, 0)),
            pl.BlockSpec((Nc, C), lambda i: (0, 0)),
        ],
        out_shape=[
            jax.ShapeDtypeStruct((Nr, C), _F32),
            jax.ShapeDtypeStruct((Nc, C), _F32),
        ],
        compiler_params=pltpu.CompilerParams(dimension_semantics=("arbitrary",)),
    )(B, xs, xd)


def _cheby_fused(Ls, srcs, wt, R=256, cache_L=True):
    n_ops, n_src = len(Ls), len(srcs)
    N = Ls[0].shape[0]
    C = srcs[0].shape[1]
    W = C * n_src
    n_slots = wt.shape[0] // 128
    m = (n_slots - 1) // n_ops
    C_OUT = wt.shape[1]
    nR = N // R

    def body(*refs):
        L_refs = refs[:n_ops]
        src_refs = refs[n_ops:n_ops + n_src]
        wt_ref = refs[n_ops + n_src]
        y_ref = refs[n_ops + n_src + 1]
        Lbs = refs[n_ops + n_src + 2:n_ops + n_src + 2 + n_ops] if cache_L else None
        chain = refs[-1]
        p = pl.program_id(0)
        i = pl.program_id(1)

        @pl.when((p == 0) & (i == 0))
        def _():
            chain[...] = jnp.zeros_like(chain)
            for s in range(n_src):
                chain[:, s * C:(s + 1) * C] = _b(src_refs[s][...])

        rows = pl.ds(i * R, R)

        if cache_L:
            @pl.when(p == 0)
            def _():
                for o in range(n_ops):
                    blk = _b(L_refs[o][...])
                    Lbs[o][rows, :] = blk
                    sl = 1 + o * m
                    chain[rows, sl * 128:sl * 128 + W] = _b(
                        jnp.dot(blk, chain[:, 0:W], preferred_element_type=_F32))

            @pl.when(p > 0)
            def _():
                for o in range(n_ops):
                    src = chain[:, pl.ds((o * m + p) * 128, W)]
                    chain[rows, pl.ds((1 + o * m + p) * 128, W)] = _b(
                        jnp.dot(Lbs[o][rows, :], src, preferred_element_type=_F32))
        else:
            for o in range(n_ops):
                src = chain[:, pl.ds(jnp.where(p == 0, 0, o * m + p) * 128, W)]
                chain[rows, pl.ds((1 + o * m + p) * 128, W)] = _b(
                    jnp.dot(_b(L_refs[o][...]), src, preferred_element_type=_F32))

        @pl.when(p == m - 1)
        def _():
            y_ref[rows, :] = jnp.dot(chain[rows, :], wt_ref[...],
                                     preferred_element_type=_F32)

    if cache_L:
        l_index = lambda p, i: (jnp.where(p == 0, i, 0), 0)
    else:
        l_index = lambda p, i: (i, 0)
    in_specs = (
        [pl.BlockSpec((R, N), l_index) for _ in Ls]
        + [pl.BlockSpec((N, C), lambda p, i: (0, 0)) for _ in srcs]
        + [pl.BlockSpec(wt.shape, lambda p, i: (0, 0))]
    )
    scratch = [pltpu.VMEM((N, N), _BF16) for _ in Ls] if cache_L else []
    return pl.pallas_call(
        body,
        grid=(m, nR),
        in_specs=in_specs,
        out_specs=pl.BlockSpec((N, C_OUT), lambda p, i: (0, 0)),
        out_shape=jax.ShapeDtypeStruct((N, C_OUT), _F32),
        scratch_shapes=scratch + [pltpu.VMEM((N, n_slots * 128), _BF16)],
        compiler_params=pltpu.CompilerParams(
            dimension_semantics=("arbitrary", "arbitrary"),
            vmem_limit_bytes=100 * 1024 * 1024),
    )(*Ls, *srcs, wt)


def _stack_weights(wt, n_src, C):
    K, _, C_OUT = wt.shape
    n_slots = K // n_src
    w = wt.reshape(n_src, n_slots, C, C_OUT).transpose(1, 0, 2, 3)
    w = w.reshape(n_slots, n_src * C, C_OUT)
    w = jnp.pad(w, ((0, 0), (0, 128 - n_src * C), (0, 0)))
    return _b(w.reshape(n_slots * 128, C_OUT))


def _layer2_node(B1, h1, h0, L0, wt, out_W, out_b):
    N1 = B1.shape[1]
    N0 = L0.shape[0]
    C = h0.shape[1]
    RB = 512
    n_slots = wt.shape[0] // 128

    def body(B1_ref, h1_ref, h0_ref, L0_ref, wt_ref, ow_ref, ob_ref,
             lg_ref, Lb0, chain, y0s):
        t = pl.program_id(0)

        @pl.when(t == 0)
        def _():
            chain[...] = jnp.zeros_like(chain)
            chain[:, 0:C] = _b(h0_ref[...])

        @pl.when(t < 2)
        def _():
            chain[pl.ds(t * RB, RB), C:2 * C] = _b(
                jnp.dot(_b(B1_ref[...]), _b(h1_ref[...]),
                        preferred_element_type=_F32))

        @pl.when((t >= 2) & (t < 4))
        def _():
            rows = pl.ds((t - 2) * RB, RB)
            blk = _b(L0_ref[...])
            Lb0[rows, :] = blk
            chain[rows, 128:128 + 2 * C] = _b(
                jnp.dot(blk, chain[:, 0:2 * C], preferred_element_type=_F32))

        @pl.when((t >= 4) & (t < 6))
        def _():
            rows = pl.ds((t - 4) * RB, RB)
            chain[rows, 256:256 + 2 * C] = _b(
                jnp.dot(Lb0[rows, :], chain[:, 128:128 + 2 * C],
                        preferred_element_type=_F32))
            y0s[rows, :] = jnp.dot(chain[rows, :], wt_ref[...],
                                   preferred_element_type=_F32)

        @pl.when(t == 6)
        def _():
            lg_ref[...] = jax.nn.sigmoid(
                jnp.dot(_b(y0s[...]), _b(ow_ref[...]),
                        preferred_element_type=_F32) + ob_ref[...])

    def l0_index(t):
        u = jnp.clip(t - 2, 0, 3)
        return (jnp.where(u < 2, u, 0), 0)

    return pl.pallas_call(
        body,
        grid=(7,),
        in_specs=[
            pl.BlockSpec((RB, N1), lambda t: (jnp.clip(t, 0, 1), 0)),
            pl.BlockSpec((N1, C), lambda t: (0, 0)),
            pl.BlockSpec((N0, C), lambda t: (0, 0)),
            pl.BlockSpec((RB, N0), l0_index),
            pl.BlockSpec(wt.shape, lambda t: (0, 0)),
            pl.BlockSpec(out_W.shape, lambda t: (0, 0)),
            pl.BlockSpec((1, out_W.shape[1]), lambda t: (0, 0)),
        ],
        out_specs=pl.BlockSpec((N0, out_W.shape[1]), lambda t: (0, 0)),
        out_shape=jax.ShapeDtypeStruct((N0, out_W.shape[1]), _F32),
        scratch_shapes=[pltpu.VMEM((N0, N0), _BF16),
                        pltpu.VMEM((N0, n_slots * 128), _BF16),
                        pltpu.VMEM((N0, C), _F32)],
        compiler_params=pltpu.CompilerParams(
            dimension_semantics=("arbitrary",),
            vmem_limit_bytes=100 * 1024 * 1024),
    )(B1, h1, h0, L0, wt, out_W, out_b.reshape(1, -1))


def _logits_body(h_ref, w_ref, b_ref, o_ref):
    o_ref[...] = jax.nn.sigmoid(
        jnp.dot(_b(h_ref[...]), _b(w_ref[...]), preferred_element_type=_F32) + b_ref[...])


def _logits(h, W, b):
    return pl.pallas_call(
        _logits_body,
        out_shape=jax.ShapeDtypeStruct((h.shape[0], W.shape[1]), _F32),
    )(h, W, b.reshape(1, -1))


def kernel(x_0, x_1, x_2, laplacian_0, laplacian_down_1, laplacian_up_1,
           laplacian_2, incidence_1, incidence_2, in_W0, in_b0, in_W1, in_b1,
           in_W2, in_b2, w0_l0, w1_l0, w2_l0, w0_l1, w1_l1, w2_l1,
           out_W, out_b):
    h0, h1, h2 = _embed(x_0, x_1, x_2, in_W0, in_b0, in_W1, in_b1, in_W2, in_b2)

    t01, t10 = _incidence(incidence_1, h1, h0, R=512)
    t12 = _incidence_fwd(incidence_2, h2, R=512)
    y0 = _cheby_fused([laplacian_0], [h0, t01],
                      _stack_weights(jnp.transpose(w0_l0, (2, 0, 1)), 2, 32),
                      R=512)
    y1 = _cheby_fused([laplacian_down_1, laplacian_up_1], [h1, t10, t12],
                      _stack_weights(jnp.transpose(w1_l0, (2, 0, 1)), 3, 32),
                      R=128)
    h0, h1 = y0, y1

    return _layer2_node(incidence_1, h1, h0, laplacian_0,
                        _stack_weights(jnp.transpose(w0_l1, (2, 0, 1)), 2, 32),
                        out_W, out_b)


# speedup vs baseline: 1.1816x; 1.1816x over previous
import jax
import jax.numpy as jnp
from jax.experimental import pallas as pl
from jax.experimental.pallas import tpu as pltpu

_F32 = jnp.float32
_BF16 = jnp.bfloat16


def _b(v):
    return v.astype(_BF16)


def _embed_body(x0, x1, x2, w0, b0, w1, b1, w2, b2, h0, h1, h2):
    h0[...] = jnp.dot(_b(x0[...]), _b(w0[...]), preferred_element_type=_F32) + b0[...]
    h1[...] = jnp.dot(_b(x1[...]), _b(w1[...]), preferred_element_type=_F32) + b1[...]
    h2[...] = jnp.dot(_b(x2[...]), _b(w2[...]), preferred_element_type=_F32) + b2[...]


def _embed(x0, x1, x2, W0, b0, W1, b1, W2, b2):
    C = W0.shape[1]
    outs = [jax.ShapeDtypeStruct((x.shape[0], C), _F32) for x in (x0, x1, x2)]
    return pl.pallas_call(_embed_body, out_shape=outs)(
        x0, x1, x2, W0, b0.reshape(1, -1), W1, b1.reshape(1, -1), W2, b2.reshape(1, -1)
    )


def _inc_body(B_ref, xs_ref, xd_ref, f_ref, bwd_ref):
    i = pl.program_id(0)
    blk = _b(B_ref[...])
    f_ref[...] = jnp.dot(blk, _b(xs_ref[...]), preferred_element_type=_F32)

    @pl.when(i == 0)
    def _():
        bwd_ref[...] = jnp.zeros_like(bwd_ref)

    bwd_ref[...] += jax.lax.dot_general(
        blk, _b(xd_ref[...]), dimension_numbers=(((0,), (0,)), ((), ())),
        preferred_element_type=_F32)


def _inc_fwd_body(B_ref, xs_ref, f_ref):
    f_ref[...] = jnp.dot(_b(B_ref[...]), _b(xs_ref[...]),
                         preferred_element_type=_F32)


def _incidence_fwd(B, xs, R=512):
    Nr, Nc = B.shape
    C = xs.shape[1]
    return pl.pallas_call(
        _inc_fwd_body,
        grid=(Nr // R,),
        in_specs=[
            pl.BlockSpec((R, Nc), lambda i: (i, 0)),
            pl.BlockSpec((Nc, C), lambda i: (0, 0)),
        ],
        out_specs=pl.BlockSpec((R, C), lambda i: (i, 0)),
        out_shape=jax.ShapeDtypeStruct((Nr, C), _F32),
        compiler_params=pltpu.CompilerParams(dimension_semantics=("arbitrary",)),
    )(B, xs)


def _incidence(B, xs, xd, R=256):
    Nr, Nc = B.shape
    C = xs.shape[1]
    return pl.pallas_call(
        _inc_body,
        grid=(Nr // R,),
        in_specs=[
            pl.BlockSpec((R, Nc), lambda i: (i, 0)),
            pl.BlockSpec((Nc, C), lambda i: (0, 0)),
            pl.BlockSpec((R, C), lambda i: (i, 0)),
        ],
        out_specs=[
            pl.BlockSpec((R, C), lambda i: (i, 0)),
            pl.BlockSpec((Nc, C), lambda i: (0, 0)),
        ],
        out_shape=[
            jax.ShapeDtypeStruct((Nr, C), _F32),
            jax.ShapeDtypeStruct((Nc, C), _F32),
        ],
        compiler_params=pltpu.CompilerParams(dimension_semantics=("arbitrary",)),
    )(B, xs, xd)


def _cheby_fused(Ls, srcs, wt, R=256, cache_L=True):
    n_ops, n_src = len(Ls), len(srcs)
    N = Ls[0].shape[0]
    C = srcs[0].shape[1]
    W = C * n_src
    n_slots = wt.shape[0] // 128
    m = (n_slots - 1) // n_ops
    C_OUT = wt.shape[1]
    nR = N // R

    def body(*refs):
        L_refs = refs[:n_ops]
        src_refs = refs[n_ops:n_ops + n_src]
        wt_ref = refs[n_ops + n_src]
        y_ref = refs[n_ops + n_src + 1]
        Lbs = refs[n_ops + n_src + 2:n_ops + n_src + 2 + n_ops] if cache_L else None
        chain = refs[-1]
        p = pl.program_id(0)
        i = pl.program_id(1)

        @pl.when((p == 0) & (i == 0))
        def _():
            chain[...] = jnp.zeros_like(chain)
            for s in range(n_src):
                chain[:, s * C:(s + 1) * C] = _b(src_refs[s][...])

        rows = pl.ds(i * R, R)

        if cache_L:
            @pl.when(p == 0)
            def _():
                for o in range(n_ops):
                    blk = _b(L_refs[o][...])
                    Lbs[o][rows, :] = blk
                    sl = 1 + o * m
                    chain[rows, sl * 128:sl * 128 + W] = _b(
                        jnp.dot(blk, chain[:, 0:W], preferred_element_type=_F32))

            @pl.when(p > 0)
            def _():
                for o in range(n_ops):
                    src = chain[:, pl.ds((o * m + p) * 128, W)]
                    chain[rows, pl.ds((1 + o * m + p) * 128, W)] = _b(
                        jnp.dot(Lbs[o][rows, :], src, preferred_element_type=_F32))
        else:
            for o in range(n_ops):
                src = chain[:, pl.ds(jnp.where(p == 0, 0, o * m + p) * 128, W)]
                chain[rows, pl.ds((1 + o * m + p) * 128, W)] = _b(
                    jnp.dot(_b(L_refs[o][...]), src, preferred_element_type=_F32))

        @pl.when(p == m - 1)
        def _():
            y_ref[rows, :] = jnp.dot(chain[rows, :], wt_ref[...],
                                     preferred_element_type=_F32)

    if cache_L:
        l_index = lambda p, i: (jnp.where(p == 0, i, 0), 0)
    else:
        l_index = lambda p, i: (i, 0)
    in_specs = (
        [pl.BlockSpec((R, N), l_index) for _ in Ls]
        + [pl.BlockSpec((N, C), lambda p, i: (0, 0)) for _ in srcs]
        + [pl.BlockSpec(wt.shape, lambda p, i: (0, 0))]
    )
    scratch = [pltpu.VMEM((N, N), _BF16) for _ in Ls] if cache_L else []
    return pl.pallas_call(
        body,
        grid=(m, nR),
        in_specs=in_specs,
        out_specs=pl.BlockSpec((N, C_OUT), lambda p, i: (0, 0)),
        out_shape=jax.ShapeDtypeStruct((N, C_OUT), _F32),
        scratch_shapes=scratch + [pltpu.VMEM((N, n_slots * 128), _BF16)],
        compiler_params=pltpu.CompilerParams(
            dimension_semantics=("arbitrary", "arbitrary"),
            vmem_limit_bytes=100 * 1024 * 1024),
    )(*Ls, *srcs, wt)


def _edge_horner(Ld, Lu, srcs, wt, R=256):
    N = Ld.shape[0]
    C = srcs[0].shape[1]
    n_src = len(srcs)
    W = C * n_src
    C_OUT = wt.shape[1]
    nR = N // R

    def body(s0, s1, s2, Ld_ref, Lu_ref, wt_ref, y_ref,
             Lbd, Lbu, X, Id, Md1, Md2, Mu1, Mu2, G1d, G1u):
        p = pl.program_id(0)
        i = pl.program_id(1)

        @pl.when((p == 0) & (i == 0))
        def _():
            for s, sref in enumerate((s0, s1, s2)):
                X[:, s * C:(s + 1) * C] = _b(sref[...])
            xv = X[...]
            Id[...] = _b(jnp.dot(xv, wt_ref[0:W], preferred_element_type=_F32))
            Md1[...] = _b(jnp.dot(xv, wt_ref[128:128 + W],
                                  preferred_element_type=_F32))
            Md2[...] = _b(jnp.dot(xv, wt_ref[256:256 + W],
                                  preferred_element_type=_F32))
            Mu1[...] = _b(jnp.dot(xv, wt_ref[384:384 + W],
                                  preferred_element_type=_F32))
            Mu2[...] = _b(jnp.dot(xv, wt_ref[512:512 + W],
                                  preferred_element_type=_F32))

        rows = pl.ds(i * R, R)

        @pl.when(p == 0)
        def _():
            blkd = _b(Ld_ref[...])
            Lbd[rows, :] = blkd
            G1d[rows, :] = _b(jnp.dot(blkd, Md2[...],
                                      preferred_element_type=_F32))
            blku = _b(Lu_ref[...])
            Lbu[rows, :] = blku
            G1u[rows, :] = _b(jnp.dot(blku, Mu2[...],
                                      preferred_element_type=_F32))

        @pl.when((p == 1) & (i == 0))
        def _():
            Md1[...] = Md1[...] + G1d[...]
            Mu1[...] = Mu1[...] + G1u[...]

        @pl.when(p == 1)
        def _():
            y_ref[rows, :] = (
                Id[rows, :].astype(_F32)
                + jnp.dot(Lbd[rows, :], Md1[...], preferred_element_type=_F32)
                + jnp.dot(Lbu[rows, :], Mu1[...], preferred_element_type=_F32))

    mk = lambda shape, dt: pltpu.VMEM(shape, dt)
    l_index = lambda p, i: (jnp.where(p == 0, i, 0), 0)
    return pl.pallas_call(
        body,
        grid=(2, nR),
        in_specs=(
            [pl.BlockSpec((N, C), lambda p, i: (0, 0)) for _ in srcs]
            + [pl.BlockSpec((R, N), l_index), pl.BlockSpec((R, N), l_index)]
            + [pl.BlockSpec(wt.shape, lambda p, i: (0, 0))]
        ),
        out_specs=pl.BlockSpec((N, C_OUT), lambda p, i: (0, 0)),
        out_shape=jax.ShapeDtypeStruct((N, C_OUT), _F32),
        scratch_shapes=[
            mk((N, N), _BF16), mk((N, N), _BF16), mk((N, W), _BF16),
            mk((N, C_OUT), _BF16), mk((N, C_OUT), _BF16), mk((N, C_OUT), _BF16),
            mk((N, C_OUT), _BF16), mk((N, C_OUT), _BF16), mk((N, C_OUT), _BF16),
            mk((N, C_OUT), _BF16),
        ],
        compiler_params=pltpu.CompilerParams(
            dimension_semantics=("arbitrary", "arbitrary"),
            vmem_limit_bytes=100 * 1024 * 1024),
    )(*srcs, Ld, Lu, wt)


def _stack_weights(wt, n_src, C):
    K, _, C_OUT = wt.shape
    n_slots = K // n_src
    w = wt.reshape(n_src, n_slots, C, C_OUT).transpose(1, 0, 2, 3)
    w = w.reshape(n_slots, n_src * C, C_OUT)
    w = jnp.pad(w, ((0, 0), (0, 128 - n_src * C), (0, 0)))
    return _b(w.reshape(n_slots * 128, C_OUT))


def _layer2_node(B1, h1, h0, L0, wt, out_W, out_b):
    N1 = B1.shape[1]
    N0 = L0.shape[0]
    C = h0.shape[1]
    RB = 512
    n_slots = wt.shape[0] // 128

    def body(B1_ref, h1_ref, h0_ref, L0_ref, wt_ref, ow_ref, ob_ref,
             lg_ref, Lb0, chain, y0s):
        t = pl.program_id(0)

        @pl.when(t == 0)
        def _():
            chain[...] = jnp.zeros_like(chain)
            chain[:, 0:C] = _b(h0_ref[...])

        @pl.when(t < 2)
        def _():
            chain[pl.ds(t * RB, RB), C:2 * C] = _b(
                jnp.dot(_b(B1_ref[...]), _b(h1_ref[...]),
                        preferred_element_type=_F32))

        @pl.when((t >= 2) & (t < 4))
        def _():
            rows = pl.ds((t - 2) * RB, RB)
            blk = _b(L0_ref[...])
            Lb0[rows, :] = blk
            chain[rows, 128:128 + 2 * C] = _b(
                jnp.dot(blk, chain[:, 0:2 * C], preferred_element_type=_F32))

        @pl.when((t >= 4) & (t < 6))
        def _():
            rows = pl.ds((t - 4) * RB, RB)
            chain[rows, 256:256 + 2 * C] = _b(
                jnp.dot(Lb0[rows, :], chain[:, 128:128 + 2 * C],
                        preferred_element_type=_F32))
            y0s[rows, :] = jnp.dot(chain[rows, :], wt_ref[...],
                                   preferred_element_type=_F32)

        @pl.when(t == 6)
        def _():
            lg_ref[...] = jax.nn.sigmoid(
                jnp.dot(_b(y0s[...]), _b(ow_ref[...]),
                        preferred_element_type=_F32) + ob_ref[...])

    def l0_index(t):
        u = jnp.clip(t - 2, 0, 3)
        return (jnp.where(u < 2, u, 0), 0)

    return pl.pallas_call(
        body,
        grid=(7,),
        in_specs=[
            pl.BlockSpec((RB, N1), lambda t: (jnp.clip(t, 0, 1), 0)),
            pl.BlockSpec((N1, C), lambda t: (0, 0)),
            pl.BlockSpec((N0, C), lambda t: (0, 0)),
            pl.BlockSpec((RB, N0), l0_index),
            pl.BlockSpec(wt.shape, lambda t: (0, 0)),
            pl.BlockSpec(out_W.shape, lambda t: (0, 0)),
            pl.BlockSpec((1, out_W.shape[1]), lambda t: (0, 0)),
        ],
        out_specs=pl.BlockSpec((N0, out_W.shape[1]), lambda t: (0, 0)),
        out_shape=jax.ShapeDtypeStruct((N0, out_W.shape[1]), _F32),
        scratch_shapes=[pltpu.VMEM((N0, N0), _BF16),
                        pltpu.VMEM((N0, n_slots * 128), _BF16),
                        pltpu.VMEM((N0, C), _F32)],
        compiler_params=pltpu.CompilerParams(
            dimension_semantics=("arbitrary",),
            vmem_limit_bytes=100 * 1024 * 1024),
    )(B1, h1, h0, L0, wt, out_W, out_b.reshape(1, -1))


def _logits_body(h_ref, w_ref, b_ref, o_ref):
    o_ref[...] = jax.nn.sigmoid(
        jnp.dot(_b(h_ref[...]), _b(w_ref[...]), preferred_element_type=_F32) + b_ref[...])


def _logits(h, W, b):
    return pl.pallas_call(
        _logits_body,
        out_shape=jax.ShapeDtypeStruct((h.shape[0], W.shape[1]), _F32),
    )(h, W, b.reshape(1, -1))


def kernel(x_0, x_1, x_2, laplacian_0, laplacian_down_1, laplacian_up_1,
           laplacian_2, incidence_1, incidence_2, in_W0, in_b0, in_W1, in_b1,
           in_W2, in_b2, w0_l0, w1_l0, w2_l0, w0_l1, w1_l1, w2_l1,
           out_W, out_b):
    h0, h1, h2 = _embed(x_0, x_1, x_2, in_W0, in_b0, in_W1, in_b1, in_W2, in_b2)

    t01, t10 = _incidence(incidence_1, h1, h0, R=512)
    t12 = _incidence_fwd(incidence_2, h2, R=512)
    y0 = _cheby_fused([laplacian_0], [h0, t01],
                      _stack_weights(jnp.transpose(w0_l0, (2, 0, 1)), 2, 32),
                      R=512)
    y1 = _edge_horner(laplacian_down_1, laplacian_up_1, [h1, t10, t12],
                      _stack_weights(jnp.transpose(w1_l0, (2, 0, 1)), 3, 32),
                      R=256)
    h0, h1 = y0, y1

    return _layer2_node(incidence_1, h1, h0, laplacian_0,
                        _stack_weights(jnp.transpose(w0_l1, (2, 0, 1)), 2, 32),
                        out_W, out_b)


# speedup vs baseline: 1.1827x; 1.0009x over previous
import jax
import jax.numpy as jnp
from jax.experimental import pallas as pl
from jax.experimental.pallas import tpu as pltpu

_F32 = jnp.float32
_BF16 = jnp.bfloat16


def _b(v):
    return v.astype(_BF16)


def _embed_body(x0, x1, x2, w0, b0, w1, b1, w2, b2, h0, h1, h2):
    h0[...] = jnp.dot(_b(x0[...]), _b(w0[...]), preferred_element_type=_F32) + b0[...]
    h1[...] = jnp.dot(_b(x1[...]), _b(w1[...]), preferred_element_type=_F32) + b1[...]
    h2[...] = jnp.dot(_b(x2[...]), _b(w2[...]), preferred_element_type=_F32) + b2[...]


def _embed(x0, x1, x2, W0, b0, W1, b1, W2, b2):
    C = W0.shape[1]
    outs = [jax.ShapeDtypeStruct((x.shape[0], C), _F32) for x in (x0, x1, x2)]
    return pl.pallas_call(_embed_body, out_shape=outs)(
        x0, x1, x2, W0, b0.reshape(1, -1), W1, b1.reshape(1, -1), W2, b2.reshape(1, -1)
    )


def _inc_body(B_ref, xs_ref, xd_ref, f_ref, bwd_ref):
    i = pl.program_id(0)
    blk = _b(B_ref[...])
    f_ref[...] = jnp.dot(blk, _b(xs_ref[...]), preferred_element_type=_F32)

    @pl.when(i == 0)
    def _():
        bwd_ref[...] = jnp.zeros_like(bwd_ref)

    bwd_ref[...] += jax.lax.dot_general(
        blk, _b(xd_ref[...]), dimension_numbers=(((0,), (0,)), ((), ())),
        preferred_element_type=_F32)


def _inc_fwd_body(B_ref, xs_ref, f_ref):
    f_ref[...] = jnp.dot(_b(B_ref[...]), _b(xs_ref[...]),
                         preferred_element_type=_F32)


def _incidence_fwd(B, xs, R=512):
    Nr, Nc = B.shape
    C = xs.shape[1]
    return pl.pallas_call(
        _inc_fwd_body,
        grid=(Nr // R,),
        in_specs=[
            pl.BlockSpec((R, Nc), lambda i: (i, 0)),
            pl.BlockSpec((Nc, C), lambda i: (0, 0)),
        ],
        out_specs=pl.BlockSpec((R, C), lambda i: (i, 0)),
        out_shape=jax.ShapeDtypeStruct((Nr, C), _F32),
        compiler_params=pltpu.CompilerParams(dimension_semantics=("arbitrary",)),
    )(B, xs)


def _incidence(B, xs, xd, R=256):
    Nr, Nc = B.shape
    C = xs.shape[1]
    return pl.pallas_call(
        _inc_body,
        grid=(Nr // R,),
        in_specs=[
            pl.BlockSpec((R, Nc), lambda i: (i, 0)),
            pl.BlockSpec((Nc, C), lambda i: (0, 0)),
            pl.BlockSpec((R, C), lambda i: (i, 0)),
        ],
        out_specs=[
            pl.BlockSpec((R, C), lambda i: (i, 0)),
            pl.BlockSpec((Nc, C), lambda i: (0, 0)),
        ],
        out_shape=[
            jax.ShapeDtypeStruct((Nr, C), _F32),
            jax.ShapeDtypeStruct((Nc, C), _F32),
        ],
        compiler_params=pltpu.CompilerParams(dimension_semantics=("arbitrary",)),
    )(B, xs, xd)


def _cheby_fused(Ls, srcs, wt, R=256, cache_L=True):
    n_ops, n_src = len(Ls), len(srcs)
    N = Ls[0].shape[0]
    C = srcs[0].shape[1]
    W = C * n_src
    n_slots = wt.shape[0] // 128
    m = (n_slots - 1) // n_ops
    C_OUT = wt.shape[1]
    nR = N // R

    def body(*refs):
        L_refs = refs[:n_ops]
        src_refs = refs[n_ops:n_ops + n_src]
        wt_ref = refs[n_ops + n_src]
        y_ref = refs[n_ops + n_src + 1]
        Lbs = refs[n_ops + n_src + 2:n_ops + n_src + 2 + n_ops] if cache_L else None
        chain = refs[-1]
        p = pl.program_id(0)
        i = pl.program_id(1)

        @pl.when((p == 0) & (i == 0))
        def _():
            chain[...] = jnp.zeros_like(chain)
            for s in range(n_src):
                chain[:, s * C:(s + 1) * C] = _b(src_refs[s][...])

        rows = pl.ds(i * R, R)

        if cache_L:
            @pl.when(p == 0)
            def _():
                for o in range(n_ops):
                    blk = _b(L_refs[o][...])
                    Lbs[o][rows, :] = blk
                    sl = 1 + o * m
                    chain[rows, sl * 128:sl * 128 + W] = _b(
                        jnp.dot(blk, chain[:, 0:W], preferred_element_type=_F32))

            @pl.when(p > 0)
            def _():
                for o in range(n_ops):
                    src = chain[:, pl.ds((o * m + p) * 128, W)]
                    chain[rows, pl.ds((1 + o * m + p) * 128, W)] = _b(
                        jnp.dot(Lbs[o][rows, :], src, preferred_element_type=_F32))
        else:
            for o in range(n_ops):
                src = chain[:, pl.ds(jnp.where(p == 0, 0, o * m + p) * 128, W)]
                chain[rows, pl.ds((1 + o * m + p) * 128, W)] = _b(
                    jnp.dot(_b(L_refs[o][...]), src, preferred_element_type=_F32))

        @pl.when(p == m - 1)
        def _():
            y_ref[rows, :] = jnp.dot(chain[rows, :], wt_ref[...],
                                     preferred_element_type=_F32)

    if cache_L:
        l_index = lambda p, i: (jnp.where(p == 0, i, 0), 0)
    else:
        l_index = lambda p, i: (i, 0)
    in_specs = (
        [pl.BlockSpec((R, N), l_index) for _ in Ls]
        + [pl.BlockSpec((N, C), lambda p, i: (0, 0)) for _ in srcs]
        + [pl.BlockSpec(wt.shape, lambda p, i: (0, 0))]
    )
    scratch = [pltpu.VMEM((N, N), _BF16) for _ in Ls] if cache_L else []
    return pl.pallas_call(
        body,
        grid=(m, nR),
        in_specs=in_specs,
        out_specs=pl.BlockSpec((N, C_OUT), lambda p, i: (0, 0)),
        out_shape=jax.ShapeDtypeStruct((N, C_OUT), _F32),
        scratch_shapes=scratch + [pltpu.VMEM((N, n_slots * 128), _BF16)],
        compiler_params=pltpu.CompilerParams(
            dimension_semantics=("arbitrary", "arbitrary"),
            vmem_limit_bytes=100 * 1024 * 1024),
    )(*Ls, *srcs, wt)


def _node_horner(L, srcs, wt, R=512):
    N = L.shape[0]
    C = srcs[0].shape[1]
    W = C * len(srcs)
    C_OUT = wt.shape[1]
    nR = N // R

    def body(s0, s1, L_ref, wt_ref, y_ref, Lb, X, Id, M1, M2, G1):
        p = pl.program_id(0)
        i = pl.program_id(1)

        @pl.when((p == 0) & (i == 0))
        def _():
            for s, sref in enumerate((s0, s1)):
                X[:, s * C:(s + 1) * C] = _b(sref[...])
            xv = X[...]
            Id[...] = _b(jnp.dot(xv, wt_ref[0:W], preferred_element_type=_F32))
            M1[...] = _b(jnp.dot(xv, wt_ref[128:128 + W],
                                 preferred_element_type=_F32))
            M2[...] = _b(jnp.dot(xv, wt_ref[256:256 + W],
                                 preferred_element_type=_F32))

        rows = pl.ds(i * R, R)

        @pl.when(p == 0)
        def _():
            blk = _b(L_ref[...])
            Lb[rows, :] = blk
            G1[rows, :] = _b(jnp.dot(blk, M2[...], preferred_element_type=_F32))

        @pl.when((p == 1) & (i == 0))
        def _():
            M1[...] = M1[...] + G1[...]

        @pl.when(p == 1)
        def _():
            y_ref[rows, :] = (Id[rows, :].astype(_F32)
                              + jnp.dot(Lb[rows, :], M1[...],
                                        preferred_element_type=_F32))

    mk = lambda shape, dt: pltpu.VMEM(shape, dt)
    l_index = lambda p, i: (jnp.where(p == 0, i, 0), 0)
    return pl.pallas_call(
        body,
        grid=(2, nR),
        in_specs=(
            [pl.BlockSpec((N, C), lambda p, i: (0, 0)) for _ in srcs]
            + [pl.BlockSpec((R, N), l_index)]
            + [pl.BlockSpec(wt.shape, lambda p, i: (0, 0))]
        ),
        out_specs=pl.BlockSpec((N, C_OUT), lambda p, i: (0, 0)),
        out_shape=jax.ShapeDtypeStruct((N, C_OUT), _F32),
        scratch_shapes=[
            mk((N, N), _BF16), mk((N, W), _BF16), mk((N, C_OUT), _BF16),
            mk((N, C_OUT), _BF16), mk((N, C_OUT), _BF16), mk((N, C_OUT), _BF16),
        ],
        compiler_params=pltpu.CompilerParams(
            dimension_semantics=("arbitrary", "arbitrary"),
            vmem_limit_bytes=100 * 1024 * 1024),
    )(*srcs, L, wt)


def _edge_horner(Ld, Lu, srcs, wt, R=256):
    N = Ld.shape[0]
    C = srcs[0].shape[1]
    n_src = len(srcs)
    W = C * n_src
    C_OUT = wt.shape[1]
    nR = N // R

    def body(s0, s1, s2, Ld_ref, Lu_ref, wt_ref, y_ref,
             Lbd, Lbu, X, Id, Md1, Md2, Mu1, Mu2, G1d, G1u):
        p = pl.program_id(0)
        i = pl.program_id(1)

        @pl.when((p == 0) & (i == 0))
        def _():
            for s, sref in enumerate((s0, s1, s2)):
                X[:, s * C:(s + 1) * C] = _b(sref[...])
            xv = X[...]
            Id[...] = _b(jnp.dot(xv, wt_ref[0:W], preferred_element_type=_F32))
            Md1[...] = _b(jnp.dot(xv, wt_ref[128:128 + W],
                                  preferred_element_type=_F32))
            Md2[...] = _b(jnp.dot(xv, wt_ref[256:256 + W],
                                  preferred_element_type=_F32))
            Mu1[...] = _b(jnp.dot(xv, wt_ref[384:384 + W],
                                  preferred_element_type=_F32))
            Mu2[...] = _b(jnp.dot(xv, wt_ref[512:512 + W],
                                  preferred_element_type=_F32))

        rows = pl.ds(i * R, R)

        @pl.when(p == 0)
        def _():
            blkd = _b(Ld_ref[...])
            Lbd[rows, :] = blkd
            G1d[rows, :] = _b(jnp.dot(blkd, Md2[...],
                                      preferred_element_type=_F32))
            blku = _b(Lu_ref[...])
            Lbu[rows, :] = blku
            G1u[rows, :] = _b(jnp.dot(blku, Mu2[...],
                                      preferred_element_type=_F32))

        @pl.when((p == 1) & (i == 0))
        def _():
            Md1[...] = Md1[...] + G1d[...]
            Mu1[...] = Mu1[...] + G1u[...]

        @pl.when(p == 1)
        def _():
            y_ref[rows, :] = (
                Id[rows, :].astype(_F32)
                + jnp.dot(Lbd[rows, :], Md1[...], preferred_element_type=_F32)
                + jnp.dot(Lbu[rows, :], Mu1[...], preferred_element_type=_F32))

    mk = lambda shape, dt: pltpu.VMEM(shape, dt)
    l_index = lambda p, i: (jnp.where(p == 0, i, 0), 0)
    return pl.pallas_call(
        body,
        grid=(2, nR),
        in_specs=(
            [pl.BlockSpec((N, C), lambda p, i: (0, 0)) for _ in srcs]
            + [pl.BlockSpec((R, N), l_index), pl.BlockSpec((R, N), l_index)]
            + [pl.BlockSpec(wt.shape, lambda p, i: (0, 0))]
        ),
        out_specs=pl.BlockSpec((N, C_OUT), lambda p, i: (0, 0)),
        out_shape=jax.ShapeDtypeStruct((N, C_OUT), _F32),
        scratch_shapes=[
            mk((N, N), _BF16), mk((N, N), _BF16), mk((N, W), _BF16),
            mk((N, C_OUT), _BF16), mk((N, C_OUT), _BF16), mk((N, C_OUT), _BF16),
            mk((N, C_OUT), _BF16), mk((N, C_OUT), _BF16), mk((N, C_OUT), _BF16),
            mk((N, C_OUT), _BF16),
        ],
        compiler_params=pltpu.CompilerParams(
            dimension_semantics=("arbitrary", "arbitrary"),
            vmem_limit_bytes=100 * 1024 * 1024),
    )(*srcs, Ld, Lu, wt)


def _stack_weights(wt, n_src, C):
    K, _, C_OUT = wt.shape
    n_slots = K // n_src
    w = wt.reshape(n_src, n_slots, C, C_OUT).transpose(1, 0, 2, 3)
    w = w.reshape(n_slots, n_src * C, C_OUT)
    w = jnp.pad(w, ((0, 0), (0, 128 - n_src * C), (0, 0)))
    return _b(w.reshape(n_slots * 128, C_OUT))


def _layer2_node(B1, h1, h0, L0, wt, out_W, out_b):
    N1 = B1.shape[1]
    N0 = L0.shape[0]
    C = h0.shape[1]
    RB = 512
    n_slots = wt.shape[0] // 128

    def body(B1_ref, h1_ref, h0_ref, L0_ref, wt_ref, ow_ref, ob_ref,
             lg_ref, Lb0, chain, y0s):
        t = pl.program_id(0)

        @pl.when(t == 0)
        def _():
            chain[...] = jnp.zeros_like(chain)
            chain[:, 0:C] = _b(h0_ref[...])

        @pl.when(t < 2)
        def _():
            chain[pl.ds(t * RB, RB), C:2 * C] = _b(
                jnp.dot(_b(B1_ref[...]), _b(h1_ref[...]),
                        preferred_element_type=_F32))

        @pl.when((t >= 2) & (t < 4))
        def _():
            rows = pl.ds((t - 2) * RB, RB)
            blk = _b(L0_ref[...])
            Lb0[rows, :] = blk
            chain[rows, 128:128 + 2 * C] = _b(
                jnp.dot(blk, chain[:, 0:2 * C], preferred_element_type=_F32))

        @pl.when((t >= 4) & (t < 6))
        def _():
            rows = pl.ds((t - 4) * RB, RB)
            chain[rows, 256:256 + 2 * C] = _b(
                jnp.dot(Lb0[rows, :], chain[:, 128:128 + 2 * C],
                        preferred_element_type=_F32))
            y0s[rows, :] = jnp.dot(chain[rows, :], wt_ref[...],
                                   preferred_element_type=_F32)

        @pl.when(t == 6)
        def _():
            lg_ref[...] = jax.nn.sigmoid(
                jnp.dot(_b(y0s[...]), _b(ow_ref[...]),
                        preferred_element_type=_F32) + ob_ref[...])

    def l0_index(t):
        u = jnp.clip(t - 2, 0, 3)
        return (jnp.where(u < 2, u, 0), 0)

    return pl.pallas_call(
        body,
        grid=(7,),
        in_specs=[
            pl.BlockSpec((RB, N1), lambda t: (jnp.clip(t, 0, 1), 0)),
            pl.BlockSpec((N1, C), lambda t: (0, 0)),
            pl.BlockSpec((N0, C), lambda t: (0, 0)),
            pl.BlockSpec((RB, N0), l0_index),
            pl.BlockSpec(wt.shape, lambda t: (0, 0)),
            pl.BlockSpec(out_W.shape, lambda t: (0, 0)),
            pl.BlockSpec((1, out_W.shape[1]), lambda t: (0, 0)),
        ],
        out_specs=pl.BlockSpec((N0, out_W.shape[1]), lambda t: (0, 0)),
        out_shape=jax.ShapeDtypeStruct((N0, out_W.shape[1]), _F32),
        scratch_shapes=[pltpu.VMEM((N0, N0), _BF16),
                        pltpu.VMEM((N0, n_slots * 128), _BF16),
                        pltpu.VMEM((N0, C), _F32)],
        compiler_params=pltpu.CompilerParams(
            dimension_semantics=("arbitrary",),
            vmem_limit_bytes=100 * 1024 * 1024),
    )(B1, h1, h0, L0, wt, out_W, out_b.reshape(1, -1))


def _logits_body(h_ref, w_ref, b_ref, o_ref):
    o_ref[...] = jax.nn.sigmoid(
        jnp.dot(_b(h_ref[...]), _b(w_ref[...]), preferred_element_type=_F32) + b_ref[...])


def _logits(h, W, b):
    return pl.pallas_call(
        _logits_body,
        out_shape=jax.ShapeDtypeStruct((h.shape[0], W.shape[1]), _F32),
    )(h, W, b.reshape(1, -1))


def kernel(x_0, x_1, x_2, laplacian_0, laplacian_down_1, laplacian_up_1,
           laplacian_2, incidence_1, incidence_2, in_W0, in_b0, in_W1, in_b1,
           in_W2, in_b2, w0_l0, w1_l0, w2_l0, w0_l1, w1_l1, w2_l1,
           out_W, out_b):
    h0, h1, h2 = _embed(x_0, x_1, x_2, in_W0, in_b0, in_W1, in_b1, in_W2, in_b2)

    t01, t10 = _incidence(incidence_1, h1, h0, R=512)
    t12 = _incidence_fwd(incidence_2, h2, R=512)
    y0 = _node_horner(laplacian_0, [h0, t01],
                      _stack_weights(jnp.transpose(w0_l0, (2, 0, 1)), 2, 32),
                      R=512)
    y1 = _edge_horner(laplacian_down_1, laplacian_up_1, [h1, t10, t12],
                      _stack_weights(jnp.transpose(w1_l0, (2, 0, 1)), 3, 32),
                      R=256)
    h0, h1 = y0, y1

    return _layer2_node(incidence_1, h1, h0, laplacian_0,
                        _stack_weights(jnp.transpose(w0_l1, (2, 0, 1)), 2, 32),
                        out_W, out_b)
